# baseline (speedup 1.0000x reference)
import functools

import jax
import jax.numpy as jnp
from jax import lax
from jax.experimental import pallas as pl
from jax.experimental.pallas import tpu as pltpu
from jax.experimental.pallas import tpu_sc as plsc

NC = 2
NS = 16
NW = NC * NS

CH = 128
K = 4

VB = 16384


def _widen(table_t, vp):
    d, v = table_t.shape

    def body(in_ref, out_ref):
        tt = jnp.transpose(in_ref[...], (1, 0))
        tp = tt.reshape(VB // 2, 2, d)
        pad = jnp.zeros((VB // 2, 64 - d), dtype=tt.dtype)
        out_ref[...] = jnp.concatenate(
            [tp[:, 0, :], pad, tp[:, 1, :], pad], axis=1
        )

    return pl.pallas_call(
        body,
        grid=(pl.cdiv(vp, VB),),
        in_specs=[pl.BlockSpec((d, VB), lambda i: (0, i))],
        out_specs=pl.BlockSpec((VB // 2, 128), lambda i: (i, 0)),
        out_shape=jax.ShapeDtypeStruct((vp // 2, 128), jnp.float32),
    )(table_t)


def _gather_t(table128, idxp, b, p, d):
    n = b * p
    nchunks = n // CH
    chunks_per_p = b // CH
    mesh = plsc.VectorSubcoreMesh(
        core_axis_name="c", subcore_axis_name="s", num_cores=NC, num_subcores=NS
    )
    chunks_per_w = nchunks // NW
    iters = chunks_per_w // K

    @functools.partial(
        pl.kernel,
        out_type=jax.ShapeDtypeStruct((p, d, b), jnp.float32),
        mesh=mesh,
        scratch_types=[
            pltpu.VMEM((K, CH), jnp.int32),
            pltpu.VMEM((K, CH), jnp.int32),
            pltpu.VMEM((K, CH, 128), jnp.float32),
            pltpu.VMEM((d, CH), jnp.float32),
            pltpu.SemaphoreType.DMA,
        ],
        compiler_params=pltpu.CompilerParams(
            use_tc_tiling_on_sc=True, needs_layout_passes=False
        ),
    )
    def body(table_hbm, idx_hbm, out_hbm, idx_v, row_v, rows_v, t_v, sem):
        wid = lax.axis_index("s") * NC + lax.axis_index("c")
        chunk0 = wid * chunks_per_w
        lane = lax.iota(jnp.int32, 16)

        def step(i, carry):
            c0 = chunk0 + i * K
            pltpu.sync_copy(idx_hbm.at[pl.ds(c0, K)], idx_v)
            copies = []
            for j in range(K):
                for r0 in range(0, CH, 16):
                    row_v[j, pl.ds(r0, 16)] = (
                        idx_v[j, pl.ds(r0, 16)] >> 1
                    )
                copies.append(
                    pltpu.async_copy(
                        table_hbm.at[row_v.at[j]],
                        rows_v.at[j],
                        sem,
                    )
                )
            for j in range(K):
                copies[j].wait()
                c = c0 + j
                pj = c // chunks_per_p
                nb0 = (c % chunks_per_p) * CH
                offs = [
                    (idx_v[j, pl.ds(r0, 16)] & 1) * 64
                    for r0 in range(0, CH, 16)
                ]

                def word(w, carry2):
                    for g, r0 in enumerate(range(0, CH, 16)):
                        vals = plsc.load_gather(
                            rows_v.at[j], [lane + r0, offs[g] + w]
                        )
                        t_v[w, pl.ds(r0, 16)] = vals
                    return carry2

                lax.fori_loop(0, d, word, 0)
                pltpu.sync_copy(
                    t_v,
                    out_hbm.at[pj, :, pl.ds(pl.multiple_of(nb0, CH), CH)],
                )
            return carry

        lax.fori_loop(0, iters, step, 0)

    return body(table128, idxp)


def kernel(indices, table):
    b, p = indices.shape
    v, d = table.shape
    vp = (v + 127) // 128 * 128
    table128 = _widen(table.T, vp)
    idxp = indices.T.reshape(b * p // CH, CH).astype(jnp.int32)
    out_t = _gather_t(table128, idxp, b, p, d)
    return jnp.transpose(out_t, (2, 0, 1))


# speedup vs baseline: 1.3027x; 1.3027x over previous
import functools

import jax
import jax.numpy as jnp
from jax import lax
from jax.experimental import pallas as pl
from jax.experimental.pallas import tpu as pltpu
from jax.experimental.pallas import tpu_sc as plsc

NC = 2
NS = 16
NW = NC * NS

CH = 128
K = 4

VB = 32768


def _widen(table_t, vp):
    d, v = table_t.shape

    def body(in_ref, out_ref):
        tt = jnp.transpose(in_ref[...], (1, 0))
        pad = jnp.zeros((VB, 128 - d), dtype=tt.dtype)
        out_ref[...] = jnp.concatenate([tt, pad], axis=1)

    return pl.pallas_call(
        body,
        grid=(pl.cdiv(vp, VB),),
        in_specs=[pl.BlockSpec((d, VB), lambda i: (0, i))],
        out_specs=pl.BlockSpec((VB, 128), lambda i: (i, 0)),
        out_shape=jax.ShapeDtypeStruct((vp, 128), jnp.float32),
    )(table_t)


def _gather_t(table128, idxp, b, p, d):
    n = b * p
    nchunks = n // CH
    chunks_per_p = b // CH
    mesh = plsc.VectorSubcoreMesh(
        core_axis_name="c", subcore_axis_name="s", num_cores=NC, num_subcores=NS
    )
    chunks_per_w = nchunks // NW
    iters = chunks_per_w // K

    @functools.partial(
        pl.kernel,
        out_type=jax.ShapeDtypeStruct((p, d, b), jnp.float32),
        mesh=mesh,
        scratch_types=[
            pltpu.VMEM((K, CH), jnp.int32),
            pltpu.VMEM((K, CH), jnp.int32),
            pltpu.VMEM((K, CH, 128), jnp.float32),
            pltpu.VMEM((d, CH), jnp.float32),
            pltpu.SemaphoreType.DMA,
            pltpu.SemaphoreType.DMA,
        ],
        compiler_params=pltpu.CompilerParams(
            use_tc_tiling_on_sc=True, needs_layout_passes=False
        ),
    )
    def body(table_hbm, idx_hbm, out_hbm, idx_a, idx_b, rows_v, t_v, sem, sem_st):
        wid = lax.axis_index("s") * NC + lax.axis_index("c")
        chunk0 = wid * chunks_per_w
        lane = lax.iota(jnp.int32, 16)

        pltpu.sync_copy(idx_hbm.at[pl.ds(chunk0, K)], idx_a)
        for j in range(K):
            pltpu.async_copy(
                table_hbm.at[idx_a.at[j]], rows_v.at[j], sem
            )

        def step(i, carry):
            c0 = chunk0 + i * K
            even = i % 2 == 0
            more = i + 1 < iters

            @pl.when(more & even)
            def _():
                pltpu.sync_copy(idx_hbm.at[pl.ds(c0 + K, K)], idx_b)

            @pl.when(more & jnp.logical_not(even))
            def _():
                pltpu.sync_copy(idx_hbm.at[pl.ds(c0 + K, K)], idx_a)

            for j in range(K):
                pltpu.make_async_copy(
                    table_hbm.at[pl.ds(0, CH)], rows_v.at[j], sem
                ).wait()
                c = c0 + j
                pj = c // chunks_per_p
                nb0 = (c % chunks_per_p) * CH

                def word(w, carry2):
                    wv = jnp.full((16,), 0, jnp.int32) + w
                    for r0 in range(0, CH, 16):
                        vals = plsc.load_gather(
                            rows_v.at[j], [lane + r0, wv]
                        )
                        t_v[w, pl.ds(r0, 16)] = vals
                    return carry2

                lax.fori_loop(0, d, word, 0)
                pltpu.sync_copy(
                    t_v,
                    out_hbm.at[pj, :, pl.ds(pl.multiple_of(nb0, CH), CH)],
                )

                @pl.when(more & even)
                def _():
                    pltpu.async_copy(
                        table_hbm.at[idx_b.at[j]], rows_v.at[j], sem
                    )

                @pl.when(more & jnp.logical_not(even))
                def _():
                    pltpu.async_copy(
                        table_hbm.at[idx_a.at[j]], rows_v.at[j], sem
                    )
            return carry

        lax.fori_loop(0, iters, step, 0)

    return body(table128, idxp)


def kernel(indices, table):
    b, p = indices.shape
    v, d = table.shape
    vp = (v + 127) // 128 * 128
    table128 = _widen(table.T, vp)
    idxp = indices.T.reshape(b * p // CH, CH).astype(jnp.int32)
    out_t = _gather_t(table128, idxp, b, p, d)
    return jnp.transpose(out_t, (2, 0, 1))
